# baseline (speedup 1.0000x reference)
import functools

import jax
import jax.numpy as jnp
from jax import lax
from jax.experimental import pallas as pl
from jax.experimental.pallas import tpu as pltpu
from jax.experimental.pallas import tpu_sc as plsc

R = 64
C = 16
B = 4
S = 1024
N = B * S
STD = 0.6 / 64
INV_STD2 = float(1.0 / (STD * STD))

_CNT_OFFS = (
    (-2, -2, 16.0),
    (-2, -1, 16.0),
    (-2, 0, 16.0),
    (-2, 1, 16.0),
    (-2, 2, 16.0),
    (-1, -2, 16.0),
    (-1, -1, 16.0),
    (-1, 0, 13.0),
)


def _prep_body(pos_ref, x_ref, pw_ref, pb_ref, vx_ref, vy_ref, vz_ref,
               ga_ref, gb_ref):
    x2 = x_ref[...]
    pos2 = pos_ref[...]
    z = jnp.dot(x2, pw_ref[...], preferred_element_type=jnp.float32)
    corr = 0.1 * jax.nn.sigmoid(z + pb_ref[...])
    p = (pos2 + corr) * float(R)
    fr = p - jnp.round(p)
    g = jnp.exp(-jnp.sum(fr * fr, axis=1) * INV_STD2)
    ci = lax.broadcasted_iota(jnp.int32, (N, C), 1)
    sa = jnp.sum(jnp.where(ci < 8, x2, 0.0), axis=1)
    sb = jnp.sum(jnp.where(ci >= 8, x2, 0.0), axis=1)
    vi = p.astype(jnp.int32)
    di = lax.broadcasted_iota(jnp.int32, (N, 3), 1)
    vx_ref[...] = jnp.sum(jnp.where(di == 0, vi, 0), axis=1)
    vy_ref[...] = jnp.sum(jnp.where(di == 1, vi, 0), axis=1)
    vz_ref[...] = jnp.sum(jnp.where(di == 2, vi, 0), axis=1)
    ga_ref[...] = g * sa
    gb_ref[...] = g * sb


def _prep(pos2, x2, pw, pb_row):
    return pl.pallas_call(
        _prep_body,
        out_shape=(
            jax.ShapeDtypeStruct((N,), jnp.int32),
            jax.ShapeDtypeStruct((N,), jnp.int32),
            jax.ShapeDtypeStruct((N,), jnp.int32),
            jax.ShapeDtypeStruct((N,), jnp.float32),
            jax.ShapeDtypeStruct((N,), jnp.float32),
        ),
    )(pos2, x2, pw, pb_row)


SLAB = 8 * R * R


def _scatter_body(vx_hbm, vy_hbm, vz_hbm, ga_hbm, gb_hbm,
                  cnt_hbm, s7_hbm, s8_hbm,
                  vxv, vyv, vzv, gav, gbv, cnt, s7, s8):
    wid = lax.axis_index("c") * 16 + lax.axis_index("s")
    b = wid >> 3
    x0 = (wid & 7) * 8
    base = b * S

    zeros = jnp.zeros((16,), jnp.float32)

    def _zero(i, _):
        sl = pl.ds(i * 16, 16)
        cnt[sl] = zeros
        s7[sl] = zeros
        s8[sl] = zeros
        return 0

    lax.fori_loop(0, SLAB // 16, _zero, 0)

    pltpu.sync_copy(vx_hbm.at[pl.ds(base, S)], vxv)
    pltpu.sync_copy(vy_hbm.at[pl.ds(base, S)], vyv)
    pltpu.sync_copy(vz_hbm.at[pl.ds(base, S)], vzv)
    pltpu.sync_copy(ga_hbm.at[pl.ds(base, S)], gav)
    pltpu.sync_copy(gb_hbm.at[pl.ds(base, S)], gbv)

    w16 = jnp.full((16,), 16.0, jnp.float32)
    w13 = jnp.full((16,), 13.0, jnp.float32)

    def _clip(v):
        return jnp.minimum(jnp.maximum(v, 0), R - 1)

    def _point_block(i, _):
        sl = pl.ds(i * 16, 16)
        vx = vxv[sl]
        vy = vyv[sl]
        vz = vzv[sl]
        ga = gav[sl]
        gb = gbv[sl]

        cxa = _clip(vx - 2) - x0
        cxb = _clip(vx - 1) - x0
        m_a = (cxa >= 0) & (cxa < 8)
        m_b = (cxb >= 0) & (cxb < 8)
        row_c = _clip(vy - 2) * R
        zs = [_clip(vz - 2), _clip(vz - 1), _clip(vz), _clip(vz + 1),
              _clip(vz + 2)]

        for dx, dz, w in _CNT_OFFS:
            relx = cxa if dx == -2 else cxb
            m = m_a if dx == -2 else m_b
            wv = w16 if w == 16.0 else w13
            lin = relx * (R * R) + row_c + zs[dz + 2]
            plsc.addupdate_scatter(cnt, [lin], wv, mask=m)

        cxc = _clip(vx) - x0
        m_v = (cxc >= 0) & (cxc < 8)
        linc = cxc * (R * R) + _clip(vy) * R + zs[2]
        plsc.addupdate_scatter(s7, [linc], ga, mask=m_v)
        plsc.addupdate_scatter(s8, [linc], gb, mask=m_v)
        return 0

    lax.fori_loop(0, S // 16, _point_block, 0)

    dst = pl.ds(x0 * R * R, SLAB)
    pltpu.sync_copy(cnt, cnt_hbm.at[b, dst])
    pltpu.sync_copy(s7, s7_hbm.at[b, dst])
    pltpu.sync_copy(s8, s8_hbm.at[b, dst])


def _scatter(vx, vy, vz, ga, gb):
    mesh = plsc.VectorSubcoreMesh(core_axis_name="c", subcore_axis_name="s")
    grid_t = jax.ShapeDtypeStruct((B, R * R * R), jnp.float32)
    fn = functools.partial(
        pl.kernel,
        mesh=mesh,
        out_type=(grid_t, grid_t, grid_t),
        compiler_params=pltpu.CompilerParams(needs_layout_passes=False),
        scratch_types=[
            pltpu.VMEM((S,), jnp.int32),
            pltpu.VMEM((S,), jnp.int32),
            pltpu.VMEM((S,), jnp.int32),
            pltpu.VMEM((S,), jnp.float32),
            pltpu.VMEM((S,), jnp.float32),
            pltpu.VMEM((SLAB,), jnp.float32),
            pltpu.VMEM((SLAB,), jnp.float32),
            pltpu.VMEM((SLAB,), jnp.float32),
        ],
    )(_scatter_body)
    return fn(vx, vy, vz, ga, gb)


def _expand_body(cnt_ref, s7_ref, s8_ref, w16_ref, w7_ref, w8_ref, cb_ref,
                 out_ref):
    n0 = cnt_ref[...].reshape(512, R)
    a7 = s7_ref[...].reshape(512, R)
    a8 = s8_ref[...].reshape(512, R)
    y = jnp.dot(n0, w16_ref[...], preferred_element_type=jnp.float32)
    y += jnp.dot(a7, w7_ref[...], preferred_element_type=jnp.float32)
    y += jnp.dot(a8, w8_ref[...], preferred_element_type=jnp.float32)
    y += cb_ref[...]
    out_ref[...] = y.reshape(1, 8, R, R * C)


def _expand(cnt, s7, s8, w16z, w7z, w8z, cbz):
    grid_spec = pl.BlockSpec((1, 8, R, R), lambda b, xc: (b, xc, 0, 0))
    w_spec = pl.BlockSpec((R, R * C), lambda b, xc: (0, 0))
    return pl.pallas_call(
        _expand_body,
        grid=(B, R // 8),
        in_specs=[grid_spec, grid_spec, grid_spec, w_spec, w_spec, w_spec,
                  pl.BlockSpec((1, R * C), lambda b, xc: (0, 0))],
        out_specs=pl.BlockSpec((1, 8, R, R * C), lambda b, xc: (b, xc, 0, 0)),
        out_shape=jax.ShapeDtypeStruct((B, R, R, R * C), jnp.float32),
    )(cnt, s7, s8, w16z, w7z, w8z, cbz)


def kernel(positions, x, poca_w, poca_b, conv_w, conv_b):
    pos2 = positions.reshape(N, 3)
    x2 = x.reshape(N, C)
    pb_row = poca_b.reshape(1, 3)

    vx, vy, vz, ga, gb = _prep(pos2, x2, poca_w, pb_row)
    cnt, s7, s8 = _scatter(vx, vy, vz, ga, gb)
    cnt = cnt.reshape(B, R, R, R)
    s7 = s7.reshape(B, R, R, R)
    s8 = s8.reshape(B, R, R, R)

    eye = jnp.eye(R, dtype=jnp.float32)[:, :, None]
    w16z = (eye * conv_w[16][None, None, :]).reshape(R, R * C)
    w7z = (eye * conv_w[7][None, None, :]).reshape(R, R * C)
    w8z = (eye * conv_w[8][None, None, :]).reshape(R, R * C)
    cbz = jnp.tile(conv_b, R).reshape(1, R * C)

    out = _expand(cnt, s7, s8, w16z, w7z, w8z, cbz)
    return out.reshape(B, R, R, R, C)


# speedup vs baseline: 1.1554x; 1.1554x over previous
import functools

import jax
import jax.numpy as jnp
from jax import lax
from jax.experimental import pallas as pl
from jax.experimental.pallas import tpu as pltpu
from jax.experimental.pallas import tpu_sc as plsc

R = 64
C = 16
B = 4
S = 1024
N = B * S
STD = 0.6 / 64
INV_STD2 = float(1.0 / (STD * STD))

_CNT_OFFS = (
    (-2, -2, 16.0),
    (-2, -1, 16.0),
    (-2, 0, 16.0),
    (-2, 1, 16.0),
    (-2, 2, 16.0),
    (-1, -2, 16.0),
    (-1, -1, 16.0),
    (-1, 0, 13.0),
)


def _prep_body(pos_ref, x_ref, pw_ref, pb_ref, vx_ref, vy_ref, vz_ref,
               ga_ref, gb_ref):
    x2 = x_ref[...]
    pos2 = pos_ref[...]
    z = jnp.dot(x2, pw_ref[...], preferred_element_type=jnp.float32)
    corr = 0.1 * jax.nn.sigmoid(z + pb_ref[...])
    p = (pos2 + corr) * float(R)
    fr = p - jnp.round(p)
    g = jnp.exp(-jnp.sum(fr * fr, axis=1) * INV_STD2)
    ci = lax.broadcasted_iota(jnp.int32, (N, C), 1)
    sa = jnp.sum(jnp.where(ci < 8, x2, 0.0), axis=1)
    sb = jnp.sum(jnp.where(ci >= 8, x2, 0.0), axis=1)
    vi = p.astype(jnp.int32)
    di = lax.broadcasted_iota(jnp.int32, (N, 3), 1)
    vx_ref[...] = jnp.sum(jnp.where(di == 0, vi, 0), axis=1)
    vy_ref[...] = jnp.sum(jnp.where(di == 1, vi, 0), axis=1)
    vz_ref[...] = jnp.sum(jnp.where(di == 2, vi, 0), axis=1)
    ga_ref[...] = g * sa
    gb_ref[...] = g * sb


def _prep(pos2, x2, pw, pb_row):
    return pl.pallas_call(
        _prep_body,
        out_shape=(
            jax.ShapeDtypeStruct((N,), jnp.int32),
            jax.ShapeDtypeStruct((N,), jnp.int32),
            jax.ShapeDtypeStruct((N,), jnp.int32),
            jax.ShapeDtypeStruct((N,), jnp.float32),
            jax.ShapeDtypeStruct((N,), jnp.float32),
        ),
    )(pos2, x2, pw, pb_row)


SLAB = 8 * R * R


def _scatter_body(vx_hbm, vy_hbm, vz_hbm, ga_hbm, gb_hbm,
                  cnt_hbm, s7_hbm, s8_hbm,
                  vxv, vyv, vzv, gav, gbv, cnt, s7, s8):
    wid = lax.axis_index("c") * 16 + lax.axis_index("s")
    b = wid >> 3
    x0 = (wid & 7) * 8
    base = b * S

    zeros = jnp.zeros((16,), jnp.float32)

    def _zero(i, _):
        sl = pl.ds(i * 16, 16)
        cnt[sl] = zeros
        s7[sl] = zeros
        s8[sl] = zeros
        return 0

    lax.fori_loop(0, SLAB // 16, _zero, 0)

    pltpu.sync_copy(vx_hbm.at[pl.ds(base, S)], vxv)
    pltpu.sync_copy(vy_hbm.at[pl.ds(base, S)], vyv)
    pltpu.sync_copy(vz_hbm.at[pl.ds(base, S)], vzv)
    pltpu.sync_copy(ga_hbm.at[pl.ds(base, S)], gav)
    pltpu.sync_copy(gb_hbm.at[pl.ds(base, S)], gbv)

    w16 = jnp.full((16,), 16.0, jnp.float32)
    w13 = jnp.full((16,), 13.0, jnp.float32)

    def _clip(v):
        return jnp.minimum(jnp.maximum(v, 0), R - 1)

    def _point_block(i, _):
        sl = pl.ds(i * 16, 16)
        vx = vxv[sl]
        vy = vyv[sl]
        vz = vzv[sl]
        ga = gav[sl]
        gb = gbv[sl]

        cxa = _clip(vx - 2) - x0
        cxb = _clip(vx - 1) - x0
        m_a = (cxa >= 0) & (cxa < 8)
        m_b = (cxb >= 0) & (cxb < 8)
        row_c = _clip(vy - 2) * R
        zs = [_clip(vz - 2), _clip(vz - 1), _clip(vz), _clip(vz + 1),
              _clip(vz + 2)]

        for dx, dz, w in _CNT_OFFS:
            relx = cxa if dx == -2 else cxb
            m = m_a if dx == -2 else m_b
            wv = w16 if w == 16.0 else w13
            lin = relx * (R * R) + row_c + zs[dz + 2]
            plsc.addupdate_scatter(cnt, [lin], wv, mask=m)

        cxc = _clip(vx) - x0
        m_v = (cxc >= 0) & (cxc < 8)
        linc = cxc * (R * R) + _clip(vy) * R + zs[2]
        plsc.addupdate_scatter(s7, [linc], ga, mask=m_v)
        plsc.addupdate_scatter(s8, [linc], gb, mask=m_v)
        return 0

    lax.fori_loop(0, S // 16, _point_block, 0)

    dst = pl.ds(x0 * R * R, SLAB)
    pltpu.sync_copy(cnt, cnt_hbm.at[b, dst])
    pltpu.sync_copy(s7, s7_hbm.at[b, dst])
    pltpu.sync_copy(s8, s8_hbm.at[b, dst])


def _scatter(vx, vy, vz, ga, gb):
    mesh = plsc.VectorSubcoreMesh(core_axis_name="c", subcore_axis_name="s")
    grid_t = jax.ShapeDtypeStruct((B, R * R * R), jnp.float32)
    fn = functools.partial(
        pl.kernel,
        mesh=mesh,
        out_type=(grid_t, grid_t, grid_t),
        compiler_params=pltpu.CompilerParams(needs_layout_passes=False),
        scratch_types=[
            pltpu.VMEM((S,), jnp.int32),
            pltpu.VMEM((S,), jnp.int32),
            pltpu.VMEM((S,), jnp.int32),
            pltpu.VMEM((S,), jnp.float32),
            pltpu.VMEM((S,), jnp.float32),
            pltpu.VMEM((SLAB,), jnp.float32),
            pltpu.VMEM((SLAB,), jnp.float32),
            pltpu.VMEM((SLAB,), jnp.float32),
        ],
    )(_scatter_body)
    return fn(vx, vy, vz, ga, gb)


def _expand_body(cnt_ref, s7_ref, s8_ref, w16_ref, w7_ref, w8_ref, cb_ref,
                 out_ref):
    n0 = cnt_ref[...].reshape(512, R)
    a7 = s7_ref[...].reshape(512, R)
    a8 = s8_ref[...].reshape(512, R)
    y = jnp.dot(n0, w16_ref[...], preferred_element_type=jnp.float32)
    y += jnp.dot(a7, w7_ref[...], preferred_element_type=jnp.float32)
    y += jnp.dot(a8, w8_ref[...], preferred_element_type=jnp.float32)
    y += cb_ref[...]
    out_ref[...] = y.reshape(1, 8, R, R * C)


def _expand(cnt, s7, s8, w16z, w7z, w8z, cbz):
    grid_spec = pl.BlockSpec((1, 8, R, R), lambda b, xc: (b, xc, 0, 0))
    w_spec = pl.BlockSpec((R, R * C), lambda b, xc: (0, 0))
    return pl.pallas_call(
        _expand_body,
        grid=(B, R // 8),
        in_specs=[grid_spec, grid_spec, grid_spec, w_spec, w_spec, w_spec,
                  pl.BlockSpec((1, R * C), lambda b, xc: (0, 0))],
        out_specs=pl.BlockSpec((1, 8, R, R * C), lambda b, xc: (b, xc, 0, 0)),
        out_shape=jax.ShapeDtypeStruct((B, R, R, R * C), jnp.float32),
    )(cnt, s7, s8, w16z, w7z, w8z, cbz)


def kernel(positions, x, poca_w, poca_b, conv_w, conv_b):
    pos2 = positions.reshape(N, 3)
    x2 = x.reshape(N, C)
    pb_row = poca_b.reshape(1, 3)

    vx, vy, vz, ga, gb = _prep(pos2, x2, poca_w, pb_row)
    cnt, s7, s8 = _scatter(vx, vy, vz, ga, gb)
    cnt = cnt.reshape(B, R, R, R)
    s7 = s7.reshape(B, R, R, R)
    s8 = s8.reshape(B, R, R, R)

    eye = jnp.eye(R, dtype=jnp.float32)[:, None, :]
    w16z = (eye * conv_w[16][None, :, None]).reshape(R, R * C)
    w7z = (eye * conv_w[7][None, :, None]).reshape(R, R * C)
    w8z = (eye * conv_w[8][None, :, None]).reshape(R, R * C)
    cbz = jnp.repeat(conv_b, R).reshape(1, R * C)

    out = _expand(cnt, s7, s8, w16z, w7z, w8z, cbz)
    return out.reshape(B, R, R, C, R).transpose(0, 1, 2, 4, 3)
